# baseline (speedup 1.0000x reference)
import functools

import jax
import jax.numpy as jnp
from jax import lax
from jax.experimental import pallas as pl
from jax.experimental.pallas import tpu as pltpu
from jax.experimental.pallas import tpu_sc as plsc

_B = 16384
_D = 64
_LANES = 16
_NU = 1000000
_C = 64


def _gmf_kernel(b_per_w, user_idx, item_idx, ut1, it1, w_vec, bias, out_hbm,
                idx_u_v, idx_i_v, off_u, off_i, dbuf_u, dbuf_i,
                w_v, b_v, out_v, sem0, sem1):
    n_ichunks = b_per_w // 128
    n_chunks = b_per_w // _C
    wid = lax.axis_index("s") * 2 + lax.axis_index("c")
    base = wid * b_per_w

    for j in range(n_ichunks):
        pltpu.sync_copy(user_idx.at[pl.ds(base + j * 128, 128)], idx_u_v.at[j])
        pltpu.sync_copy(item_idx.at[pl.ds(base + j * 128, 128)], idx_i_v.at[j])
    pltpu.sync_copy(w_vec, w_v)
    pltpu.sync_copy(bias, b_v)

    sems = [sem0, sem1]
    pats = [(lax.iota(jnp.int32, _LANES) + k * _LANES) * _NU
            for k in range(_D // _LANES)]

    def fire_chunk(c, slot):
        for g in range(_C // _LANES):
            pos = c * _C + g * _LANES
            ru = idx_u_v[pos // 128, pl.ds(pos % 128, _LANES)]
            ri = idx_i_v[pos // 128, pl.ds(pos % 128, _LANES)]
            for e in range(_LANES):
                el = g * _LANES + e
                for k in range(_D // _LANES):
                    col = (el % 2) * _D + k * _LANES
                    off_u[slot, el // 2, pl.ds(col, _LANES)] = pats[k] + ru[e]
                    off_i[slot, el // 2, pl.ds(col, _LANES)] = pats[k] + ri[e]
        for m in range(_C // 2):
            pltpu.async_copy(ut1.at[off_u.at[slot, m]],
                             dbuf_u.at[slot, pl.ds(m * 128, 128)], sems[slot])
            pltpu.async_copy(it1.at[off_i.at[slot, m]],
                             dbuf_i.at[slot, pl.ds(m * 128, 128)], sems[slot])

    def drain(slot):
        pltpu.make_async_copy(ut1.at[pl.ds(0, _C * _D)],
                              dbuf_u.at[slot], sems[slot]).wait()
        pltpu.make_async_copy(it1.at[pl.ds(0, _C * _D)],
                              dbuf_i.at[slot], sems[slot]).wait()

    w_chunks = [w_v[pl.ds(k * _LANES, _LANES)] for k in range(_D // _LANES)]
    lane = lax.iota(jnp.int32, _LANES)
    bias_vec = b_v[...]

    def compute_chunk(c, slot):
        for g in range(_C // _LANES):
            res = jnp.zeros((_LANES,), jnp.float32)
            for e in range(_LANES):
                el = g * _LANES + e
                acc = (dbuf_u[slot, pl.ds(el * _D, _LANES)]
                       * dbuf_i[slot, pl.ds(el * _D, _LANES)] * w_chunks[0])
                for k in range(1, _D // _LANES):
                    acc = acc + (dbuf_u[slot, pl.ds(el * _D + k * _LANES,
                                                    _LANES)]
                                 * dbuf_i[slot, pl.ds(el * _D + k * _LANES,
                                                      _LANES)]
                                 * w_chunks[k])
                res = jnp.where(lane == e, jnp.sum(acc), res)
            x = res + bias_vec
            out_v[pl.ds(c * _C + g * _LANES, _LANES)] = 1.0 / (1.0 +
                                                               jnp.exp(-x))

    fire_chunk(0, 0)

    def pipe_body(h, carry):
        c = h * 2
        fire_chunk(c + 1, 1)
        drain(0)
        compute_chunk(c, 0)

        @pl.when(c + 2 < n_chunks)
        def _():
            fire_chunk(c + 2, 0)

        drain(1)
        compute_chunk(c + 1, 1)
        return carry

    lax.fori_loop(0, n_chunks // 2, pipe_body, 0)

    pltpu.sync_copy(out_v, out_hbm.at[pl.ds(base, b_per_w)])


def kernel(user_input, item_input, user_table, item_table, W, b):
    info = plsc.get_sparse_core_info()
    num_workers = info.num_cores * info.num_subcores
    b_per_w = _B // num_workers
    n_ichunks = b_per_w // 128

    mesh = plsc.VectorSubcoreMesh(core_axis_name="c", subcore_axis_name="s")
    run = pl.kernel(
        functools.partial(_gmf_kernel, b_per_w),
        mesh=mesh,
        compiler_params=pltpu.CompilerParams(
            needs_layout_passes=False, use_tc_tiling_on_sc=False),
        out_type=jax.ShapeDtypeStruct((_B,), jnp.float32),
        scratch_types=[
            pltpu.VMEM((n_ichunks, 128), jnp.int32),
            pltpu.VMEM((n_ichunks, 128), jnp.int32),
            pltpu.VMEM((2, _C // 2, 128), jnp.int32),
            pltpu.VMEM((2, _C // 2, 128), jnp.int32),
            pltpu.VMEM((2, _C * _D), jnp.float32),
            pltpu.VMEM((2, _C * _D), jnp.float32),
            pltpu.VMEM((_D,), jnp.float32),
            pltpu.VMEM((_LANES,), jnp.float32),
            pltpu.VMEM((b_per_w,), jnp.float32),
            pltpu.SemaphoreType.DMA,
            pltpu.SemaphoreType.DMA,
        ],
    )
    out = run(user_input.astype(jnp.int32), item_input.astype(jnp.int32),
              user_table.T.reshape(-1), item_table.T.reshape(-1),
              W.reshape(_D), jnp.broadcast_to(b.reshape(1), (_LANES,)))
    return out.reshape(_B, 1)


# speedup vs baseline: 9.1267x; 9.1267x over previous
import functools

import jax
import jax.numpy as jnp
from jax import lax
from jax.experimental import pallas as pl
from jax.experimental.pallas import tpu as pltpu
from jax.experimental.pallas import tpu_sc as plsc

_B = 16384
_D = 64
_LANES = 16
_NU = 1000000


def _gmf_kernel(b_per_w, user_idx, item_idx, user_table,
                item_table, w_vec, bias, out_hbm,
                idx_u_v, idx_i_v, rows_u, rows_i, w_v, b_v, out_v, sem):
    n_chunks = b_per_w // 128
    wid = lax.axis_index("s") * 2 + lax.axis_index("c")
    base = wid * b_per_w

    for j in range(n_chunks):
        pltpu.sync_copy(user_idx.at[pl.ds(base + j * 128, 128)], idx_u_v.at[j])
        pltpu.sync_copy(item_idx.at[pl.ds(base + j * 128, 128)], idx_i_v.at[j])
    pltpu.sync_copy(w_vec, w_v)
    pltpu.sync_copy(bias, b_v)

    copies = []
    for j in range(n_chunks):
        copies.append(pltpu.async_copy(
            user_table.at[idx_u_v.at[j]], rows_u.at[pl.ds(j * 128, 128)], sem))
        copies.append(pltpu.async_copy(
            item_table.at[idx_i_v.at[j]], rows_i.at[pl.ds(j * 128, 128)], sem))
    for c in copies:
        c.wait()

    w_chunks = [w_v[pl.ds(k * _LANES, _LANES)] for k in range(_D // _LANES)]
    lane = lax.iota(jnp.int32, _LANES)
    bias_vec = b_v[...]

    def group_body(g, carry):
        res = jnp.zeros((_LANES,), jnp.float32)
        for e in range(_LANES):
            idx = g * _LANES + e
            acc = (rows_u[idx, pl.ds(0, _LANES)]
                   * rows_i[idx, pl.ds(0, _LANES)] * w_chunks[0])
            for k in range(1, _D // _LANES):
                acc = acc + (rows_u[idx, pl.ds(k * _LANES, _LANES)]
                             * rows_i[idx, pl.ds(k * _LANES, _LANES)]
                             * w_chunks[k])
            res = jnp.where(lane == e, jnp.sum(acc), res)
        x = res + bias_vec
        out_v[pl.ds(g * _LANES, _LANES)] = 1.0 / (1.0 + jnp.exp(-x))
        return carry

    lax.fori_loop(0, b_per_w // _LANES, group_body, 0)

    pltpu.sync_copy(out_v, out_hbm.at[pl.ds(base, b_per_w)])


def kernel(user_input, item_input, user_table, item_table, W, b):
    info = plsc.get_sparse_core_info()
    num_workers = info.num_cores * info.num_subcores
    b_per_w = _B // num_workers
    n_chunks = b_per_w // 128

    mesh = plsc.VectorSubcoreMesh(core_axis_name="c", subcore_axis_name="s")
    run = pl.kernel(
        functools.partial(_gmf_kernel, b_per_w),
        mesh=mesh,
        compiler_params=pltpu.CompilerParams(
            needs_layout_passes=False, use_tc_tiling_on_sc=False),
        out_type=jax.ShapeDtypeStruct((_B,), jnp.float32),
        scratch_types=[
            pltpu.VMEM((n_chunks, 128), jnp.int32),
            pltpu.VMEM((n_chunks, 128), jnp.int32),
            pltpu.VMEM((b_per_w, _D), jnp.float32),
            pltpu.VMEM((b_per_w, _D), jnp.float32),
            pltpu.VMEM((_D,), jnp.float32),
            pltpu.VMEM((_LANES,), jnp.float32),
            pltpu.VMEM((b_per_w,), jnp.float32),
            pltpu.SemaphoreType.DMA,
        ],
    )
    mid_u = lax.optimization_barrier(user_table.reshape(_NU // 2, 2 * _D))
    mid_i = lax.optimization_barrier(item_table.reshape(_NU // 2, 2 * _D))
    ut = mid_u.reshape(_NU, _D)
    it = mid_i.reshape(_NU, _D)
    out = run(user_input.astype(jnp.int32), item_input.astype(jnp.int32),
              ut, it, W.reshape(_D),
              jnp.broadcast_to(b.reshape(1), (_LANES,)))
    return out.reshape(_B, 1)
